# Initial kernel scaffold; baseline (speedup 1.0000x reference)
import functools

import jax
import jax.numpy as jnp
from jax import lax
from jax.experimental import pallas as pl
from jax.experimental.pallas import tpu as pltpu
from jax.experimental.pallas import tpu_sc as plsc

N = 10000
E = 320000
D = 128

NC = 2
NS = 16
NW = NC * NS
EPW = E // NW
CH = 80
NCHUNK = EPW // CH
RPT = N // NS

_MESH = plsc.VectorSubcoreMesh(core_axis_name="c", subcore_axis_name="s")


@functools.partial(
    pl.kernel,
    out_type=jax.ShapeDtypeStruct((NC * N, 16), jnp.float32),
    mesh=_MESH,
    scratch_types=[
        pltpu.VMEM_SHARED((N, 16), jnp.float32),
        pltpu.VMEM((CH,), jnp.int32),
        pltpu.VMEM((CH, 16), jnp.float32),
    ],
)
def _deg_kernel(dst_hbm, ones_hbm, zeros16_hbm, out_hbm, acc, idx_v, ones_v):
    c = lax.axis_index("c")
    s = lax.axis_index("s")
    wid = s * NC + c
    pltpu.sync_copy(ones_hbm, ones_v)
    pltpu.sync_copy(
        zeros16_hbm.at[pl.ds(s * RPT, RPT)], acc.at[pl.ds(s * RPT, RPT)]
    )
    plsc.subcore_barrier()

    def body(i, carry):
        base = wid * EPW + i * CH
        pltpu.sync_copy(dst_hbm.at[pl.ds(base, CH)], idx_v)
        pltpu.sync_copy(ones_v, acc.at[idx_v], add=True)
        return carry

    lax.fori_loop(0, NCHUNK, body, 0)
    plsc.subcore_barrier()
    pltpu.sync_copy(
        acc.at[pl.ds(s * RPT, RPT)], out_hbm.at[pl.ds(c * N + s * RPT, RPT)]
    )


@functools.partial(
    pl.kernel,
    out_type=jax.ShapeDtypeStruct((NC * N, D), jnp.float32),
    mesh=_MESH,
    scratch_types=[
        pltpu.VMEM_SHARED((N, D), jnp.float32),
        pltpu.VMEM((CH,), jnp.int32),
        pltpu.VMEM((CH,), jnp.int32),
        pltpu.VMEM((CH, D), jnp.float32),
        pltpu.SemaphoreType.DMA,
    ],
)
def _agg_kernel(y_hbm, src_hbm, dst_hbm, zeros_hbm, out_hbm,
                acc, s_idx, d_idx, rows, sem):
    c = lax.axis_index("c")
    s = lax.axis_index("s")
    wid = s * NC + c

    @pl.when(c == 0)
    def _():
        pltpu.sync_copy(
            y_hbm.at[pl.ds(s * RPT, RPT)], acc.at[pl.ds(s * RPT, RPT)]
        )

    @pl.when(c != 0)
    def _():
        pltpu.sync_copy(
            zeros_hbm.at[pl.ds(s * RPT, RPT)], acc.at[pl.ds(s * RPT, RPT)]
        )

    plsc.subcore_barrier()

    def body(i, carry):
        base = wid * EPW + i * CH
        pltpu.sync_copy(src_hbm.at[pl.ds(base, CH)], s_idx)
        pltpu.sync_copy(dst_hbm.at[pl.ds(base, CH)], d_idx)
        pltpu.async_copy(y_hbm.at[s_idx], rows, sem).wait()
        pltpu.sync_copy(rows, acc.at[d_idx], add=True)
        return carry

    lax.fori_loop(0, NCHUNK, body, 0)
    plsc.subcore_barrier()
    pltpu.sync_copy(
        acc.at[pl.ds(s * RPT, RPT)], out_hbm.at[pl.ds(c * N + s * RPT, RPT)]
    )


_BLK = 1000
_GRID = N // _BLK


def _k1_body(x_ref, w_ref, d0_ref, d1_ref, y_ref, dinv_ref):
    deg = d0_ref[:, 0:1] + d1_ref[:, 0:1] + 1.0
    dinv = lax.rsqrt(deg)
    dinv_ref[...] = dinv
    xw = jnp.dot(x_ref[...], w_ref[...], preferred_element_type=jnp.float32)
    y_ref[...] = xw * dinv


def _k2_body(a0_ref, a1_ref, dinv_ref, b1_ref, w2_ref, y2_ref):
    dinv = dinv_ref[...]
    h = jnp.maximum(dinv * (a0_ref[...] + a1_ref[...]) + b1_ref[...], 0.0)
    hw = jnp.dot(h, w2_ref[...], preferred_element_type=jnp.float32)
    y2_ref[...] = hw * dinv


def _k3_body(a0_ref, a1_ref, dinv_ref, b2_ref, out_ref):
    out_ref[...] = dinv_ref[...] * (a0_ref[...] + a1_ref[...]) + b2_ref[...]


_row_spec = pl.BlockSpec((_BLK, D), lambda i: (i, 0))
_deg_spec = pl.BlockSpec((_BLK, 16), lambda i: (i, 0))
_dinv_spec = pl.BlockSpec((_BLK, 1), lambda i: (i, 0))
_w_spec = pl.BlockSpec((D, D), lambda i: (0, 0))
_b_spec = pl.BlockSpec((1, D), lambda i: (0, 0))

_k1 = pl.pallas_call(
    _k1_body,
    grid=(_GRID,),
    in_specs=[_row_spec, _w_spec, _deg_spec, _deg_spec],
    out_specs=[_row_spec, _dinv_spec],
    out_shape=[
        jax.ShapeDtypeStruct((N, D), jnp.float32),
        jax.ShapeDtypeStruct((N, 1), jnp.float32),
    ],
)

_k2 = pl.pallas_call(
    _k2_body,
    grid=(_GRID,),
    in_specs=[_row_spec, _row_spec, _dinv_spec, _b_spec, _w_spec],
    out_specs=[_row_spec],
    out_shape=[jax.ShapeDtypeStruct((N, D), jnp.float32)],
)

_k3 = pl.pallas_call(
    _k3_body,
    grid=(_GRID,),
    in_specs=[_row_spec, _row_spec, _dinv_spec, _b_spec],
    out_specs=[_row_spec],
    out_shape=[jax.ShapeDtypeStruct((N, D), jnp.float32)],
)


def kernel(x, edge_index, W1, b1, W2, b2):
    ei = edge_index.astype(jnp.int32)
    src = ei[0]
    dst = ei[1]
    ones16 = jnp.ones((CH, 16), jnp.float32)
    zeros16 = jnp.zeros((N, 16), jnp.float32)
    zeros_nd = jnp.zeros((N, D), jnp.float32)
    b1r = b1.reshape(1, D)
    b2r = b2.reshape(1, D)

    degp = _deg_kernel(dst, ones16, zeros16)
    y1, dinv = _k1(x, W1, degp[:N], degp[N:])

    agg1 = _agg_kernel(y1, src, dst, zeros_nd)
    (y2,) = _k2(agg1[:N], agg1[N:], dinv, b1r, W2)

    agg2 = _agg_kernel(y2, src, dst, zeros_nd)
    (out,) = _k3(agg2[:N], agg2[N:], dinv, b2r)
    return out


# speedup vs baseline: 11.9494x; 11.9494x over previous
import functools

import jax
import jax.numpy as jnp
from jax import lax
from jax.experimental import pallas as pl
from jax.experimental.pallas import tpu as pltpu
from jax.experimental.pallas import tpu_sc as plsc

N = 10000
E = 320000
D = 128

NC = 2
NS = 16
NW = NC * NS
EPW = E // NW
CH = 80
NCHUNK = EPW // CH
RPT = 624
RPT_LAST = N - (NS - 1) * RPT
N_PAD = 10240
DRPT = N_PAD // NS

@functools.cache
def _sc_kernels():
    mesh = plsc.VectorSubcoreMesh(
        core_axis_name="c", subcore_axis_name="s",
        num_cores=NC, num_subcores=NS,
    )

    def per_tile_rows(s, fn):
        @pl.when(s < NS - 1)
        def _():
            fn(pl.multiple_of(s * RPT, 8), RPT)

        @pl.when(s == NS - 1)
        def _():
            fn((NS - 1) * RPT, RPT_LAST)

    @functools.partial(
        pl.kernel,
        out_type=jax.ShapeDtypeStruct((NC * N, D), jnp.float32),
        mesh=mesh,
        scratch_types=[
            pltpu.VMEM_SHARED((N, D), jnp.float32),
            pltpu.VMEM((CH,), jnp.int32),
            pltpu.VMEM((CH, D), jnp.float32),
        ],
    )
    def deg_kernel(dst_hbm, ones_hbm, zeros_hbm, out_hbm, acc, idx_v, ones_v):
        c = lax.axis_index("c")
        s = lax.axis_index("s")
        wid = s * NC + c
        pltpu.sync_copy(ones_hbm, ones_v)

        def init(base, size):
            pltpu.sync_copy(
                zeros_hbm.at[pl.ds(base, size)], acc.at[pl.ds(base, size)]
            )

        per_tile_rows(s, init)
        plsc.subcore_barrier()

        def body(i, carry):
            ebase = pl.multiple_of(wid * EPW + i * CH, 8)
            pltpu.sync_copy(dst_hbm.at[pl.ds(ebase, CH)], idx_v)
            pltpu.sync_copy(ones_v, acc.at[idx_v], add=True)
            return carry

        lax.fori_loop(0, NCHUNK, body, 0)
        plsc.subcore_barrier()

        def writeback(base, size):
            out_base = pl.multiple_of(c * N + base, 8)
            pltpu.sync_copy(
                acc.at[pl.ds(base, size)], out_hbm.at[pl.ds(out_base, size)]
            )

        per_tile_rows(s, writeback)

    @functools.partial(
        pl.kernel,
        out_type=jax.ShapeDtypeStruct((NC * N, D), jnp.float32),
        mesh=mesh,
        scratch_types=[
            pltpu.VMEM_SHARED((N, D), jnp.float32),
            pltpu.VMEM((CH,), jnp.int32),
            pltpu.VMEM((CH,), jnp.int32),
            pltpu.VMEM((CH, D), jnp.float32),
            pltpu.SemaphoreType.DMA,
        ],
    )
    def agg_kernel(y_hbm, src_hbm, dst_hbm, zeros_hbm, out_hbm,
                   acc, s_idx, d_idx, rows, sem):
        c = lax.axis_index("c")
        s = lax.axis_index("s")
        wid = s * NC + c

        def init(base, size):
            @pl.when(c == 0)
            def _():
                pltpu.sync_copy(
                    y_hbm.at[pl.ds(base, size)], acc.at[pl.ds(base, size)]
                )

            @pl.when(c != 0)
            def _():
                pltpu.sync_copy(
                    zeros_hbm.at[pl.ds(base, size)], acc.at[pl.ds(base, size)]
                )

        per_tile_rows(s, init)
        plsc.subcore_barrier()

        def body(i, carry):
            base = pl.multiple_of(wid * EPW + i * CH, 8)
            pltpu.sync_copy(src_hbm.at[pl.ds(base, CH)], s_idx)
            pltpu.sync_copy(dst_hbm.at[pl.ds(base, CH)], d_idx)
            pltpu.async_copy(y_hbm.at[s_idx], rows, sem).wait()
            pltpu.sync_copy(rows, acc.at[d_idx], add=True)
            return carry

        lax.fori_loop(0, NCHUNK, body, 0)
        plsc.subcore_barrier()

        def writeback(base, size):
            out_base = pl.multiple_of(c * N + base, 8)
            pltpu.sync_copy(
                acc.at[pl.ds(base, size)], out_hbm.at[pl.ds(out_base, size)]
            )

        per_tile_rows(s, writeback)

    return deg_kernel, agg_kernel


_BLK = 1000
_GRID = N // _BLK


def _k1_body(x_ref, w_ref, d0_ref, d1_ref, y_ref, dinv_ref):
    deg = d0_ref[...] + d1_ref[...] + 1.0
    dinv = lax.rsqrt(deg)
    dinv_ref[...] = dinv
    xw = jnp.dot(x_ref[...], w_ref[...], preferred_element_type=jnp.float32)
    y_ref[...] = xw * dinv


def _k2_body(a0_ref, a1_ref, dinv_ref, b1_ref, w2_ref, y2_ref):
    dinv = dinv_ref[...]
    h = jnp.maximum(dinv * (a0_ref[...] + a1_ref[...]) + b1_ref[...], 0.0)
    hw = jnp.dot(h, w2_ref[...], preferred_element_type=jnp.float32)
    y2_ref[...] = hw * dinv


def _k3_body(a0_ref, a1_ref, dinv_ref, b2_ref, out_ref):
    out_ref[...] = dinv_ref[...] * (a0_ref[...] + a1_ref[...]) + b2_ref[...]


_row_spec = pl.BlockSpec((_BLK, D), lambda i: (i, 0))
_dinv_spec = pl.BlockSpec((_BLK, 1), lambda i: (i, 0))
_w_spec = pl.BlockSpec((D, D), lambda i: (0, 0))
_b_spec = pl.BlockSpec((1, D), lambda i: (0, 0))

_k1 = pl.pallas_call(
    _k1_body,
    grid=(_GRID,),
    in_specs=[_row_spec, _w_spec, _dinv_spec, _dinv_spec],
    out_specs=[_row_spec, _dinv_spec],
    out_shape=[
        jax.ShapeDtypeStruct((N, D), jnp.float32),
        jax.ShapeDtypeStruct((N, 1), jnp.float32),
    ],
)

_k2 = pl.pallas_call(
    _k2_body,
    grid=(_GRID,),
    in_specs=[_row_spec, _row_spec, _dinv_spec, _b_spec, _w_spec],
    out_specs=[_row_spec],
    out_shape=[jax.ShapeDtypeStruct((N, D), jnp.float32)],
)

_k3 = pl.pallas_call(
    _k3_body,
    grid=(_GRID,),
    in_specs=[_row_spec, _row_spec, _dinv_spec, _b_spec],
    out_specs=[_row_spec],
    out_shape=[jax.ShapeDtypeStruct((N, D), jnp.float32)],
)


def kernel(x, edge_index, W1, b1, W2, b2):
    ei = edge_index.astype(jnp.int32)
    src = ei[0]
    dst = ei[1]
    zeros_nd = jnp.zeros((N, D), jnp.float32)
    ones_ch = jnp.ones((CH, D), jnp.float32)
    b1r = b1.reshape(1, D)
    b2r = b2.reshape(1, D)

    _deg_kernel, _agg_kernel = _sc_kernels()
    degp = _deg_kernel(dst, ones_ch, zeros_nd)
    y1, dinv = _k1(x, W1, degp[:N, :1], degp[N:, :1])

    agg1 = _agg_kernel(y1, src, dst, zeros_nd)
    (y2,) = _k2(agg1[:N], agg1[N:], dinv, b1r, W2)

    agg2 = _agg_kernel(y2, src, dst, zeros_nd)
    (out,) = _k3(agg2[:N], agg2[N:], dinv, b2r)
    return out


# speedup vs baseline: 24.6915x; 2.0663x over previous
import functools

import jax
import jax.numpy as jnp
from jax import lax
from jax.experimental import pallas as pl
from jax.experimental.pallas import tpu as pltpu
from jax.experimental.pallas import tpu_sc as plsc

N = 10000
E = 320000
D = 128

NC = 2
NS = 16
NW = NC * NS
CH = 128
NCH_MAIN = 78
NCH_LAST = 82
EPT = NCH_MAIN * CH
DEPTH = 3
NSLOT = 84
RPT = 624
RPT_LAST = N - (NS - 1) * RPT
N_PAD = 10240
DRPT = N_PAD // NS

@functools.cache
def _sc_kernels():
    mesh = plsc.VectorSubcoreMesh(
        core_axis_name="c", subcore_axis_name="s",
        num_cores=NC, num_subcores=NS,
    )

    def per_tile_rows(s, fn):
        @pl.when(s < NS - 1)
        def _():
            fn(pl.multiple_of(s * RPT, 8), RPT)

        @pl.when(s == NS - 1)
        def _():
            fn((NS - 1) * RPT, RPT_LAST)

    @functools.partial(
        pl.kernel,
        out_type=jax.ShapeDtypeStruct((NC * N, D), jnp.float32),
        mesh=mesh,
        scratch_types=[
            pltpu.VMEM_SHARED((N, D), jnp.float32),
            pltpu.VMEM((DEPTH, CH), jnp.int32),
            pltpu.VMEM((CH, D), jnp.float32),
            pltpu.SemaphoreType.DMA,
            pltpu.SemaphoreType.DMA,
            pltpu.SemaphoreType.DMA,
        ],
    )
    def deg_kernel(dst_hbm, ones_hbm, zeros_hbm, out_hbm, acc, d_idx, ones_v,
                   ss0, ss1, ss2):
        c = lax.axis_index("c")
        s = lax.axis_index("s")
        wid = s * NC + c
        nch = jnp.where(wid == NW - 1, NCH_LAST, NCH_MAIN)
        ssem = (ss0, ss1, ss2)
        pltpu.sync_copy(ones_hbm, ones_v)

        def init(base, size):
            pltpu.sync_copy(
                zeros_hbm.at[pl.ds(base, size)], acc.at[pl.ds(base, size)]
            )

        per_tile_rows(s, init)
        plsc.subcore_barrier()

        def load_idx(i, b):
            ebase = pl.multiple_of(wid * EPT + i * CH, 8)
            pltpu.sync_copy(dst_hbm.at[pl.ds(ebase, CH)], d_idx.at[b])

        for b in range(DEPTH - 1):
            load_idx(b, b)

        def body(j, carry):
            for k in range(DEPTH):
                i = DEPTH * j + k

                @pl.when(i < nch)
                def _():
                    pltpu.async_copy(
                        ones_v, acc.at[d_idx.at[k]], ssem[k], add=True
                    )

                ip = i + DEPTH - 1
                bp = (k + DEPTH - 1) % DEPTH

                @pl.when(ip < nch)
                def _():
                    @pl.when(ip >= DEPTH)
                    def _():
                        pltpu.make_async_copy(
                            ones_v, acc.at[d_idx.at[bp]], ssem[bp]
                        ).wait()

                    load_idx(ip, bp)

            return carry

        lax.fori_loop(0, NSLOT // DEPTH, body, 0)
        for b in range(DEPTH):
            pltpu.make_async_copy(ones_v, acc.at[d_idx.at[b]], ssem[b]).wait()
        plsc.subcore_barrier()

        def writeback(base, size):
            out_base = pl.multiple_of(c * N + base, 8)
            pltpu.sync_copy(
                acc.at[pl.ds(base, size)], out_hbm.at[pl.ds(out_base, size)]
            )

        per_tile_rows(s, writeback)

    @functools.partial(
        pl.kernel,
        out_type=jax.ShapeDtypeStruct((NC * N, D), jnp.float32),
        mesh=mesh,
        scratch_types=[
            pltpu.VMEM_SHARED((N, D), jnp.float32),
            pltpu.VMEM((DEPTH, CH), jnp.int32),
            pltpu.VMEM((DEPTH, CH), jnp.int32),
            pltpu.VMEM((DEPTH, CH, D), jnp.float32),
            pltpu.SemaphoreType.DMA,
            pltpu.SemaphoreType.DMA,
            pltpu.SemaphoreType.DMA,
            pltpu.SemaphoreType.DMA,
            pltpu.SemaphoreType.DMA,
            pltpu.SemaphoreType.DMA,
        ],
    )
    def agg_kernel(y_hbm, src_hbm, dst_hbm, zeros_hbm, out_hbm,
                   acc, s_idx, d_idx, rows,
                   gs0, gs1, gs2, ss0, ss1, ss2):
        c = lax.axis_index("c")
        s = lax.axis_index("s")
        wid = s * NC + c
        nch = jnp.where(wid == NW - 1, NCH_LAST, NCH_MAIN)
        gsem = (gs0, gs1, gs2)
        ssem = (ss0, ss1, ss2)

        def init(base, size):
            @pl.when(c == 0)
            def _():
                pltpu.sync_copy(
                    y_hbm.at[pl.ds(base, size)], acc.at[pl.ds(base, size)]
                )

            @pl.when(c != 0)
            def _():
                pltpu.sync_copy(
                    zeros_hbm.at[pl.ds(base, size)], acc.at[pl.ds(base, size)]
                )

        per_tile_rows(s, init)
        plsc.subcore_barrier()

        def issue(i, b):
            ebase = pl.multiple_of(wid * EPT + i * CH, 8)
            pltpu.sync_copy(src_hbm.at[pl.ds(ebase, CH)], s_idx.at[b])
            pltpu.sync_copy(dst_hbm.at[pl.ds(ebase, CH)], d_idx.at[b])
            pltpu.async_copy(y_hbm.at[s_idx.at[b]], rows.at[b], gsem[b])

        for b in range(DEPTH - 1):
            issue(b, b)

        def body(j, carry):
            for k in range(DEPTH):
                i = DEPTH * j + k

                @pl.when(i < nch)
                def _():
                    pltpu.make_async_copy(
                        y_hbm.at[s_idx.at[k]], rows.at[k], gsem[k]
                    ).wait()
                    pltpu.async_copy(
                        rows.at[k], acc.at[d_idx.at[k]], ssem[k], add=True
                    )

                ip = i + DEPTH - 1
                bp = (k + DEPTH - 1) % DEPTH

                @pl.when(ip < nch)
                def _():
                    @pl.when(ip >= DEPTH)
                    def _():
                        pltpu.make_async_copy(
                            rows.at[bp], acc.at[d_idx.at[bp]], ssem[bp]
                        ).wait()

                    issue(ip, bp)

            return carry

        lax.fori_loop(0, NSLOT // DEPTH, body, 0)
        for b in range(DEPTH):
            pltpu.make_async_copy(
                rows.at[b], acc.at[d_idx.at[b]], ssem[b]
            ).wait()
        plsc.subcore_barrier()

        def writeback(base, size):
            out_base = pl.multiple_of(c * N + base, 8)
            pltpu.sync_copy(
                acc.at[pl.ds(base, size)], out_hbm.at[pl.ds(out_base, size)]
            )

        per_tile_rows(s, writeback)

    return deg_kernel, agg_kernel


_BLK = 1000
_GRID = N // _BLK


def _k1_body(x_ref, w_ref, d0_ref, d1_ref, y_ref, dinv_ref):
    deg = d0_ref[...] + d1_ref[...] + 1.0
    dinv = lax.rsqrt(deg)
    dinv_ref[...] = dinv
    xw = jnp.dot(x_ref[...], w_ref[...], preferred_element_type=jnp.float32)
    y_ref[...] = xw * dinv


def _k2_body(a0_ref, a1_ref, dinv_ref, b1_ref, w2_ref, y2_ref):
    dinv = dinv_ref[...]
    h = jnp.maximum(dinv * (a0_ref[...] + a1_ref[...]) + b1_ref[...], 0.0)
    hw = jnp.dot(h, w2_ref[...], preferred_element_type=jnp.float32)
    y2_ref[...] = hw * dinv


def _k3_body(a0_ref, a1_ref, dinv_ref, b2_ref, out_ref):
    out_ref[...] = dinv_ref[...] * (a0_ref[...] + a1_ref[...]) + b2_ref[...]


_row_spec = pl.BlockSpec((_BLK, D), lambda i: (i, 0))
_dinv_spec = pl.BlockSpec((_BLK, 1), lambda i: (i, 0))
_w_spec = pl.BlockSpec((D, D), lambda i: (0, 0))
_b_spec = pl.BlockSpec((1, D), lambda i: (0, 0))

_k1 = pl.pallas_call(
    _k1_body,
    grid=(_GRID,),
    in_specs=[_row_spec, _w_spec, _dinv_spec, _dinv_spec],
    out_specs=[_row_spec, _dinv_spec],
    out_shape=[
        jax.ShapeDtypeStruct((N, D), jnp.float32),
        jax.ShapeDtypeStruct((N, 1), jnp.float32),
    ],
)

_k2 = pl.pallas_call(
    _k2_body,
    grid=(_GRID,),
    in_specs=[_row_spec, _row_spec, _dinv_spec, _b_spec, _w_spec],
    out_specs=[_row_spec],
    out_shape=[jax.ShapeDtypeStruct((N, D), jnp.float32)],
)

_k3 = pl.pallas_call(
    _k3_body,
    grid=(_GRID,),
    in_specs=[_row_spec, _row_spec, _dinv_spec, _b_spec],
    out_specs=[_row_spec],
    out_shape=[jax.ShapeDtypeStruct((N, D), jnp.float32)],
)


def kernel(x, edge_index, W1, b1, W2, b2):
    ei = edge_index.astype(jnp.int32)
    src = ei[0]
    dst = ei[1]
    zeros_nd = jnp.zeros((N, D), jnp.float32)
    ones_ch = jnp.ones((CH, D), jnp.float32)
    b1r = b1.reshape(1, D)
    b2r = b2.reshape(1, D)

    _deg_kernel, _agg_kernel = _sc_kernels()
    degp = _deg_kernel(dst, ones_ch, zeros_nd)
    y1, dinv = _k1(x, W1, degp[:N, :1], degp[N:, :1])

    agg1 = _agg_kernel(y1, src, dst, zeros_nd)
    (y2,) = _k2(agg1[:N], agg1[N:], dinv, b1r, W2)

    agg2 = _agg_kernel(y2, src, dst, zeros_nd)
    (out,) = _k3(agg2[:N], agg2[N:], dinv, b2r)
    return out


# speedup vs baseline: 26.1271x; 1.0581x over previous
import functools

import jax
import jax.numpy as jnp
from jax import lax
from jax.experimental import pallas as pl
from jax.experimental.pallas import tpu as pltpu
from jax.experimental.pallas import tpu_sc as plsc

N = 10000
E = 320000
D = 128

NC = 2
NS = 16
NW = NC * NS
CH = 128
NCH_MAIN = 78
NCH_LAST = 82
EPT = NCH_MAIN * CH
DEPTH = 3
NSLOT = 84
RPT = 624
RPT_LAST = N - (NS - 1) * RPT
N_PAD = 10240
DRPT = N_PAD // NS

@functools.cache
def _sc_kernels():
    mesh = plsc.VectorSubcoreMesh(
        core_axis_name="c", subcore_axis_name="s",
        num_cores=NC, num_subcores=NS,
    )

    def per_tile_rows(s, fn):
        @pl.when(s < NS - 1)
        def _():
            fn(pl.multiple_of(s * RPT, 8), RPT)

        @pl.when(s == NS - 1)
        def _():
            fn((NS - 1) * RPT, RPT_LAST)

    @functools.partial(
        pl.kernel,
        out_type=jax.ShapeDtypeStruct((NC * N, D), jnp.float32),
        mesh=mesh,
        scratch_types=[
            pltpu.VMEM_SHARED((N, D), jnp.float32),
            pltpu.VMEM((DEPTH, CH), jnp.int32),
            pltpu.VMEM((CH, D), jnp.float32),
            pltpu.SemaphoreType.DMA,
            pltpu.SemaphoreType.DMA,
            pltpu.SemaphoreType.DMA,
        ],
    )
    def deg_kernel(dst_hbm, ones_hbm, zeros_hbm, out_hbm, acc, d_idx, ones_v,
                   ss0, ss1, ss2):
        c = lax.axis_index("c")
        s = lax.axis_index("s")
        wid = s * NC + c
        nch = jnp.where(wid == NW - 1, NCH_LAST, NCH_MAIN)
        ssem = (ss0, ss1, ss2)
        pltpu.sync_copy(ones_hbm, ones_v)

        def init(base, size):
            pltpu.sync_copy(
                zeros_hbm.at[pl.ds(base, size)], acc.at[pl.ds(base, size)]
            )

        per_tile_rows(s, init)
        plsc.subcore_barrier()

        def load_idx(i, b):
            ebase = pl.multiple_of(wid * EPT + i * CH, 8)
            pltpu.sync_copy(dst_hbm.at[pl.ds(ebase, CH)], d_idx.at[b])

        for b in range(DEPTH - 1):
            load_idx(b, b)

        def body(j, carry):
            for k in range(DEPTH):
                i = DEPTH * j + k

                @pl.when(i < nch)
                def _():
                    pltpu.async_copy(
                        ones_v, acc.at[d_idx.at[k]], ssem[k], add=True
                    )

                ip = i + DEPTH - 1
                bp = (k + DEPTH - 1) % DEPTH

                @pl.when(ip < nch)
                def _():
                    @pl.when(ip >= DEPTH)
                    def _():
                        pltpu.make_async_copy(
                            ones_v, acc.at[d_idx.at[bp]], ssem[bp]
                        ).wait()

                    load_idx(ip, bp)

            return carry

        lax.fori_loop(0, NSLOT // DEPTH, body, 0)
        for b in range(DEPTH):
            pltpu.make_async_copy(ones_v, acc.at[d_idx.at[b]], ssem[b]).wait()
        plsc.subcore_barrier()

        def writeback(base, size):
            out_base = pl.multiple_of(c * N + base, 8)
            pltpu.sync_copy(
                acc.at[pl.ds(base, size)], out_hbm.at[pl.ds(out_base, size)]
            )

        per_tile_rows(s, writeback)

    @functools.partial(
        pl.kernel,
        out_type=jax.ShapeDtypeStruct((NC * N, D), jnp.float32),
        mesh=mesh,
        scratch_types=[
            pltpu.VMEM_SHARED((N, D), jnp.float32),
            pltpu.VMEM((DEPTH, CH), jnp.int32),
            pltpu.VMEM((DEPTH, CH), jnp.int32),
            pltpu.VMEM((DEPTH, CH, D), jnp.float32),
            pltpu.SemaphoreType.DMA,
            pltpu.SemaphoreType.DMA,
            pltpu.SemaphoreType.DMA,
            pltpu.SemaphoreType.DMA,
            pltpu.SemaphoreType.DMA,
            pltpu.SemaphoreType.DMA,
        ],
    )
    def agg_kernel(y_hbm, src_hbm, dst_hbm, zeros_hbm, out_hbm,
                   acc, s_idx, d_idx, rows,
                   gs0, gs1, gs2, ss0, ss1, ss2):
        c = lax.axis_index("c")
        s = lax.axis_index("s")
        wid = s * NC + c
        nch = jnp.where(wid == NW - 1, NCH_LAST, NCH_MAIN)
        gsem = (gs0, gs1, gs2)
        ssem = (ss0, ss1, ss2)

        def init(base, size):
            @pl.when(c == 0)
            def _():
                pltpu.sync_copy(
                    y_hbm.at[pl.ds(base, size)], acc.at[pl.ds(base, size)]
                )

            @pl.when(c != 0)
            def _():
                pltpu.sync_copy(
                    zeros_hbm.at[pl.ds(base, size)], acc.at[pl.ds(base, size)]
                )

        per_tile_rows(s, init)
        plsc.subcore_barrier()

        def issue(i, b):
            ebase = pl.multiple_of(wid * EPT + i * CH, 8)
            pltpu.sync_copy(src_hbm.at[pl.ds(ebase, CH)], s_idx.at[b])
            pltpu.sync_copy(dst_hbm.at[pl.ds(ebase, CH)], d_idx.at[b])
            pltpu.async_copy(y_hbm.at[s_idx.at[b]], rows.at[b], gsem[b])

        for b in range(DEPTH - 1):
            issue(b, b)

        def body(j, carry):
            for k in range(DEPTH):
                i = DEPTH * j + k

                @pl.when(i < nch)
                def _():
                    pltpu.make_async_copy(
                        y_hbm.at[s_idx.at[k]], rows.at[k], gsem[k]
                    ).wait()
                    pltpu.async_copy(
                        rows.at[k], acc.at[d_idx.at[k]], ssem[k], add=True
                    )

                ip = i + DEPTH - 1
                bp = (k + DEPTH - 1) % DEPTH

                @pl.when(ip < nch)
                def _():
                    @pl.when(ip >= DEPTH)
                    def _():
                        pltpu.make_async_copy(
                            rows.at[bp], acc.at[d_idx.at[bp]], ssem[bp]
                        ).wait()

                    issue(ip, bp)

            return carry

        lax.fori_loop(0, NSLOT // DEPTH, body, 0)
        for b in range(DEPTH):
            pltpu.make_async_copy(
                rows.at[b], acc.at[d_idx.at[b]], ssem[b]
            ).wait()
        plsc.subcore_barrier()

        def writeback(base, size):
            out_base = pl.multiple_of(c * N + base, 8)
            pltpu.sync_copy(
                acc.at[pl.ds(base, size)], out_hbm.at[pl.ds(out_base, size)]
            )

        per_tile_rows(s, writeback)

    return deg_kernel, agg_kernel


_BLK = 1000
_GRID = N // _BLK


def _k1_body(x_ref, w_ref, d0_ref, d1_ref, y_ref, dinv_ref):
    deg = d0_ref[...] + d1_ref[...] + 1.0
    dinv = lax.rsqrt(deg)
    dinv_ref[...] = dinv
    xw = jnp.dot(x_ref[...], w_ref[...], preferred_element_type=jnp.float32)
    y_ref[...] = xw * dinv


def _k2_body(a0_ref, a1_ref, dinv_ref, b1_ref, w2_ref, y2_ref):
    dinv = dinv_ref[...]
    h = jnp.maximum(dinv * (a0_ref[...] + a1_ref[...]) + b1_ref[...], 0.0)
    hw = jnp.dot(h, w2_ref[...], preferred_element_type=jnp.float32)
    y2_ref[...] = hw * dinv


def _k3_body(a0_ref, a1_ref, dinv_ref, b2_ref, out_ref):
    out_ref[...] = dinv_ref[...] * (a0_ref[...] + a1_ref[...]) + b2_ref[...]


_row_spec = pl.BlockSpec((_BLK, D), lambda i: (i, 0))
_par1_spec = pl.BlockSpec((_BLK, D), lambda i: (i + _GRID, 0))
_dinv_spec = pl.BlockSpec((_BLK, 1), lambda i: (i, 0))
_deg1_spec = pl.BlockSpec((_BLK, 1), lambda i: (i + _GRID, 0))
_w_spec = pl.BlockSpec((D, D), lambda i: (0, 0))
_b_spec = pl.BlockSpec((1, D), lambda i: (0, 0))

_k1 = pl.pallas_call(
    _k1_body,
    grid=(_GRID,),
    in_specs=[_row_spec, _w_spec, _dinv_spec, _deg1_spec],
    out_specs=[_row_spec, _dinv_spec],
    out_shape=[
        jax.ShapeDtypeStruct((N, D), jnp.float32),
        jax.ShapeDtypeStruct((N, 1), jnp.float32),
    ],
)

_k2 = pl.pallas_call(
    _k2_body,
    grid=(_GRID,),
    in_specs=[_row_spec, _par1_spec, _dinv_spec, _b_spec, _w_spec],
    out_specs=[_row_spec],
    out_shape=[jax.ShapeDtypeStruct((N, D), jnp.float32)],
)

_k3 = pl.pallas_call(
    _k3_body,
    grid=(_GRID,),
    in_specs=[_row_spec, _par1_spec, _dinv_spec, _b_spec],
    out_specs=[_row_spec],
    out_shape=[jax.ShapeDtypeStruct((N, D), jnp.float32)],
)


def kernel(x, edge_index, W1, b1, W2, b2):
    ei = edge_index.astype(jnp.int32)
    src = ei[0]
    dst = ei[1]
    zeros_nd = jnp.zeros((N, D), jnp.float32)
    ones_ch = jnp.ones((CH, D), jnp.float32)
    b1r = b1.reshape(1, D)
    b2r = b2.reshape(1, D)

    _deg_kernel, _agg_kernel = _sc_kernels()
    degp = _deg_kernel(dst, ones_ch, zeros_nd)
    y1, dinv = _k1(x, W1, degp[:, :1], degp[:, :1])

    agg1 = _agg_kernel(y1, src, dst, zeros_nd)
    (y2,) = _k2(agg1, agg1, dinv, b1r, W2)

    agg2 = _agg_kernel(y2, src, dst, zeros_nd)
    (out,) = _k3(agg2, agg2, dinv, b2r)
    return out


# speedup vs baseline: 28.1412x; 1.0771x over previous
import functools

import jax
import jax.numpy as jnp
from jax import lax
from jax.experimental import pallas as pl
from jax.experimental.pallas import tpu as pltpu
from jax.experimental.pallas import tpu_sc as plsc

N = 10000
E = 320000
D = 128

NC = 2
NS = 16
NW = NC * NS
CH = 128
NCH_MAIN = 78
NCH_LAST = 82
EPT = NCH_MAIN * CH
DEPTH = 3
NSLOT = 84
RPT = 624
RPT_LAST = N - (NS - 1) * RPT
N_PAD = 10240
DRPT = N_PAD // NS

@functools.cache
def _sc_kernels():
    mesh = plsc.VectorSubcoreMesh(
        core_axis_name="c", subcore_axis_name="s",
        num_cores=NC, num_subcores=NS,
    )

    def per_tile_rows(s, fn):
        @pl.when(s < NS - 1)
        def _():
            fn(pl.multiple_of(s * RPT, 8), RPT)

        @pl.when(s == NS - 1)
        def _():
            fn((NS - 1) * RPT, RPT_LAST)

    @functools.partial(
        pl.kernel,
        out_type=jax.ShapeDtypeStruct((NC * N, D), jnp.float32),
        mesh=mesh,
        scratch_types=[
            pltpu.VMEM_SHARED((N, D), jnp.float32),
            pltpu.VMEM((DEPTH, CH), jnp.int32),
            pltpu.VMEM((CH, D), jnp.float32),
            pltpu.SemaphoreType.DMA,
            pltpu.SemaphoreType.DMA,
            pltpu.SemaphoreType.DMA,
        ],
    )
    def deg_kernel(dst_hbm, ones_hbm, zeros_hbm, out_hbm, acc, d_idx, ones_v,
                   ss0, ss1, ss2):
        c = lax.axis_index("c")
        s = lax.axis_index("s")
        wid = s * NC + c
        nch = jnp.where(wid == NW - 1, NCH_LAST, NCH_MAIN)
        ssem = (ss0, ss1, ss2)
        pltpu.sync_copy(ones_hbm, ones_v)

        def init(base, size):
            pltpu.sync_copy(
                zeros_hbm.at[pl.ds(base, size)], acc.at[pl.ds(base, size)]
            )

        per_tile_rows(s, init)
        plsc.subcore_barrier()

        def load_idx(i, b):
            ebase = pl.multiple_of(wid * EPT + i * CH, 8)
            pltpu.sync_copy(dst_hbm.at[pl.ds(ebase, CH)], d_idx.at[b])

        for b in range(DEPTH - 1):
            load_idx(b, b)

        def body(j, carry):
            for k in range(DEPTH):
                i = DEPTH * j + k

                @pl.when(i < nch)
                def _():
                    pltpu.async_copy(
                        ones_v, acc.at[d_idx.at[k]], ssem[k], add=True
                    )

                ip = i + DEPTH - 1
                bp = (k + DEPTH - 1) % DEPTH

                @pl.when(ip < nch)
                def _():
                    @pl.when(ip >= DEPTH)
                    def _():
                        pltpu.make_async_copy(
                            ones_v, acc.at[d_idx.at[bp]], ssem[bp]
                        ).wait()

                    load_idx(ip, bp)

            return carry

        lax.fori_loop(0, NSLOT // DEPTH, body, 0)
        for b in range(DEPTH):
            pltpu.make_async_copy(ones_v, acc.at[d_idx.at[b]], ssem[b]).wait()
        plsc.subcore_barrier()

        def writeback(base, size):
            out_base = pl.multiple_of(c * N + base, 8)
            pltpu.sync_copy(
                acc.at[pl.ds(base, size)], out_hbm.at[pl.ds(out_base, size)]
            )

        per_tile_rows(s, writeback)

    @functools.partial(
        pl.kernel,
        out_type=jax.ShapeDtypeStruct((NC * N, D), jnp.float32),
        mesh=mesh,
        scratch_types=[
            pltpu.VMEM_SHARED((N, D), jnp.float32),
            pltpu.VMEM((DEPTH, CH), jnp.int32),
            pltpu.VMEM((DEPTH, CH), jnp.int32),
            pltpu.VMEM((DEPTH, CH, D), jnp.float32),
            pltpu.SemaphoreType.DMA,
            pltpu.SemaphoreType.DMA,
            pltpu.SemaphoreType.DMA,
            pltpu.SemaphoreType.DMA,
            pltpu.SemaphoreType.DMA,
            pltpu.SemaphoreType.DMA,
            pltpu.SemaphoreType.DMA,
        ],
    )
    def agg_kernel(y_hbm, src_hbm, dst_hbm, zeros_hbm, out_hbm,
                   acc, s_idx, d_idx, rows,
                   gs0, gs1, gs2, ss0, ss1, ss2, isem):
        c = lax.axis_index("c")
        s = lax.axis_index("s")
        wid = s * NC + c
        nch = jnp.where(wid == NW - 1, NCH_LAST, NCH_MAIN)
        gsem = (gs0, gs1, gs2)
        ssem = (ss0, ss1, ss2)

        def init(base, size):
            @pl.when(c == 0)
            def _():
                pltpu.sync_copy(
                    y_hbm.at[pl.ds(base, size)], acc.at[pl.ds(base, size)]
                )

            @pl.when(c != 0)
            def _():
                pltpu.sync_copy(
                    zeros_hbm.at[pl.ds(base, size)], acc.at[pl.ds(base, size)]
                )

        per_tile_rows(s, init)
        plsc.subcore_barrier()

        def issue(i, b):
            ebase = pl.multiple_of(wid * EPT + i * CH, 8)
            pltpu.async_copy(src_hbm.at[pl.ds(ebase, CH)], s_idx.at[b], isem)
            pltpu.async_copy(dst_hbm.at[pl.ds(ebase, CH)], d_idx.at[b], isem)
            pltpu.make_async_copy(
                src_hbm.at[pl.ds(ebase, CH)], s_idx.at[b], isem
            ).wait()
            pltpu.make_async_copy(
                dst_hbm.at[pl.ds(ebase, CH)], d_idx.at[b], isem
            ).wait()
            pltpu.async_copy(y_hbm.at[s_idx.at[b]], rows.at[b], gsem[b])

        for b in range(DEPTH - 1):
            issue(b, b)

        def body(j, carry):
            for k in range(DEPTH):
                i = DEPTH * j + k

                @pl.when(i < nch)
                def _():
                    pltpu.make_async_copy(
                        y_hbm.at[s_idx.at[k]], rows.at[k], gsem[k]
                    ).wait()
                    pltpu.async_copy(
                        rows.at[k], acc.at[d_idx.at[k]], ssem[k], add=True
                    )

                ip = i + DEPTH - 1
                bp = (k + DEPTH - 1) % DEPTH

                @pl.when(ip < nch)
                def _():
                    @pl.when(ip >= DEPTH)
                    def _():
                        pltpu.make_async_copy(
                            rows.at[bp], acc.at[d_idx.at[bp]], ssem[bp]
                        ).wait()

                    issue(ip, bp)

            return carry

        lax.fori_loop(0, NSLOT // DEPTH, body, 0)
        for b in range(DEPTH):
            pltpu.make_async_copy(
                rows.at[b], acc.at[d_idx.at[b]], ssem[b]
            ).wait()
        plsc.subcore_barrier()

        def writeback(base, size):
            out_base = pl.multiple_of(c * N + base, 8)
            pltpu.sync_copy(
                acc.at[pl.ds(base, size)], out_hbm.at[pl.ds(out_base, size)]
            )

        per_tile_rows(s, writeback)

    return deg_kernel, agg_kernel


_BLK = 1000
_GRID = N // _BLK


def _k1_body(x_ref, w_ref, d0_ref, d1_ref, y_ref, dinv_ref):
    deg = d0_ref[...] + d1_ref[...] + 1.0
    dinv = lax.rsqrt(deg)
    dinv_ref[...] = dinv
    xw = jnp.dot(x_ref[...], w_ref[...], preferred_element_type=jnp.float32)
    y_ref[...] = xw * dinv


def _k2_body(a0_ref, a1_ref, dinv_ref, b1_ref, w2_ref, y2_ref):
    dinv = dinv_ref[...]
    h = jnp.maximum(dinv * (a0_ref[...] + a1_ref[...]) + b1_ref[...], 0.0)
    hw = jnp.dot(h, w2_ref[...], preferred_element_type=jnp.float32)
    y2_ref[...] = hw * dinv


def _k3_body(a0_ref, a1_ref, dinv_ref, b2_ref, out_ref):
    out_ref[...] = dinv_ref[...] * (a0_ref[...] + a1_ref[...]) + b2_ref[...]


_row_spec = pl.BlockSpec((_BLK, D), lambda i: (i, 0))
_par1_spec = pl.BlockSpec((_BLK, D), lambda i: (i + _GRID, 0))
_dinv_spec = pl.BlockSpec((_BLK, 1), lambda i: (i, 0))
_deg1_spec = pl.BlockSpec((_BLK, 1), lambda i: (i + _GRID, 0))
_w_spec = pl.BlockSpec((D, D), lambda i: (0, 0))
_b_spec = pl.BlockSpec((1, D), lambda i: (0, 0))

_k1 = pl.pallas_call(
    _k1_body,
    grid=(_GRID,),
    in_specs=[_row_spec, _w_spec, _dinv_spec, _deg1_spec],
    out_specs=[_row_spec, _dinv_spec],
    out_shape=[
        jax.ShapeDtypeStruct((N, D), jnp.float32),
        jax.ShapeDtypeStruct((N, 1), jnp.float32),
    ],
)

_k2 = pl.pallas_call(
    _k2_body,
    grid=(_GRID,),
    in_specs=[_row_spec, _par1_spec, _dinv_spec, _b_spec, _w_spec],
    out_specs=[_row_spec],
    out_shape=[jax.ShapeDtypeStruct((N, D), jnp.float32)],
)

_k3 = pl.pallas_call(
    _k3_body,
    grid=(_GRID,),
    in_specs=[_row_spec, _par1_spec, _dinv_spec, _b_spec],
    out_specs=[_row_spec],
    out_shape=[jax.ShapeDtypeStruct((N, D), jnp.float32)],
)


def kernel(x, edge_index, W1, b1, W2, b2):
    ei = edge_index.astype(jnp.int32)
    src = ei[0]
    dst = ei[1]
    zeros_nd = jnp.zeros((N, D), jnp.float32)
    ones_ch = jnp.ones((CH, D), jnp.float32)
    b1r = b1.reshape(1, D)
    b2r = b2.reshape(1, D)

    _deg_kernel, _agg_kernel = _sc_kernels()
    degp = _deg_kernel(dst, ones_ch, zeros_nd)
    y1, dinv = _k1(x, W1, degp[:, :1], degp[:, :1])

    agg1 = _agg_kernel(y1, src, dst, zeros_nd)
    (y2,) = _k2(agg1, agg1, dinv, b1r, W2)

    agg2 = _agg_kernel(y2, src, dst, zeros_nd)
    (out,) = _k3(agg2, agg2, dinv, b2r)
    return out
